# Initial kernel scaffold; baseline (speedup 1.0000x reference)
import functools

import jax
import jax.numpy as jnp
from jax import lax
from jax.experimental import pallas as pl
from jax.experimental.pallas import tpu as pltpu
from jax.experimental.pallas import tpu_sc as plsc

MAXLEN = 200
EMBED = 32
BATCH = 4096

NC = 2
NS = 16
NW = NC * NS
LANES = 16

TOTAL = BATCH * MAXLEN
PER_W = TOTAL // NW
CHUNK = 1024
GROUP = 128
NGROUP = CHUNK // GROUP
NCHUNK = PER_W // CHUNK
HALVES = EMBED // LANES

_mesh = plsc.VectorSubcoreMesh(core_axis_name="c", subcore_axis_name="s")


@functools.partial(
    pl.kernel,
    out_type=jax.ShapeDtypeStruct((NW * NCHUNK, CHUNK, EMBED), jnp.float32),
    mesh=_mesh,
    scratch_types=[
        pltpu.VMEM((NGROUP, GROUP), jnp.int32),
        pltpu.VMEM((CHUNK, EMBED), jnp.float32),
        pltpu.VMEM((MAXLEN, EMBED), jnp.float32),
        pltpu.SemaphoreType.DMA,
    ],
)
def _embed_sc(x_hbm, tok_hbm, pos_hbm, out_hbm, idx_v, rows_v, pos_v, sem):
    wid = lax.axis_index("s") * NC + lax.axis_index("c")
    pltpu.sync_copy(pos_hbm, pos_v)

    def chunk_body(c, _):
        j = wid * NCHUNK + c
        pltpu.sync_copy(x_hbm.at[j], idx_v)
        copies = []
        for g in range(NGROUP):
            copies.append(
                pltpu.async_copy(
                    tok_hbm.at[idx_v.at[g]],
                    rows_v.at[pl.ds(g * GROUP, GROUP)],
                    sem,
                )
            )
        for cp in copies:
            cp.wait()

        p0 = lax.rem(j * CHUNK, MAXLEN)

        def row_body(r, p):
            p = jnp.where(p >= MAXLEN, p - MAXLEN, p)
            for h in range(HALVES):
                sl = pl.ds(h * LANES, LANES)
                rows_v[r, sl] = rows_v[r, sl] + pos_v[p, sl]
            return p + 1

        lax.fori_loop(0, CHUNK, row_body, p0, unroll=2)
        pltpu.sync_copy(rows_v, out_hbm.at[j])
        return 0

    lax.fori_loop(0, NCHUNK, chunk_body, 0)


def kernel(x, token_table, pos_table):
    xr = x.astype(jnp.int32).reshape(NW * NCHUNK, NGROUP, GROUP)
    out = _embed_sc(xr, token_table, pos_table)
    return out.reshape(BATCH, MAXLEN, EMBED)


# speedup vs baseline: 1.1839x; 1.1839x over previous
import functools

import jax
import jax.numpy as jnp
from jax import lax
from jax.experimental import pallas as pl
from jax.experimental.pallas import tpu as pltpu
from jax.experimental.pallas import tpu_sc as plsc

MAXLEN = 200
EMBED = 32
BATCH = 4096

NC = 2
NS = 16
NW = NC * NS
LANES = 16

TOTAL = BATCH * MAXLEN
PER_W = TOTAL // NW
CHUNK = 1024
GROUP = 128
NGROUP = CHUNK // GROUP
NCHUNK = PER_W // CHUNK
HALVES = EMBED // LANES

_mesh = plsc.VectorSubcoreMesh(core_axis_name="c", subcore_axis_name="s")


@functools.partial(
    pl.kernel,
    out_type=jax.ShapeDtypeStruct((NW * NCHUNK, CHUNK, EMBED), jnp.float32),
    mesh=_mesh,
    compiler_params=pltpu.CompilerParams(use_tc_tiling_on_sc=False),
    scratch_types=[
        pltpu.VMEM((NGROUP, GROUP), jnp.int32),
        pltpu.VMEM((CHUNK, EMBED), jnp.float32),
        pltpu.VMEM((MAXLEN, EMBED), jnp.float32),
        pltpu.SemaphoreType.DMA,
    ],
)
def _embed_sc(x_hbm, tok_hbm, pos_hbm, out_hbm, idx_v, rows_v, pos_v, sem):
    wid = lax.axis_index("s") * NC + lax.axis_index("c")
    pltpu.sync_copy(pos_hbm, pos_v)

    def chunk_body(c, _):
        j = wid * NCHUNK + c
        pltpu.sync_copy(x_hbm.at[j], idx_v)
        copies = []
        for g in range(NGROUP):
            copies.append(
                pltpu.async_copy(
                    tok_hbm.at[idx_v.at[g]],
                    rows_v.at[pl.ds(g * GROUP, GROUP)],
                    sem,
                )
            )
        for cp in copies:
            cp.wait()

        p0 = lax.rem(j * CHUNK, MAXLEN)

        def row_body(r, p):
            p = jnp.where(p >= MAXLEN, p - MAXLEN, p)
            for h in range(HALVES):
                sl = pl.ds(h * LANES, LANES)
                rows_v[r, sl] = rows_v[r, sl] + pos_v[p, sl]
            return p + 1

        lax.fori_loop(0, CHUNK, row_body, p0, unroll=2)
        pltpu.sync_copy(rows_v, out_hbm.at[j])
        return 0

    lax.fori_loop(0, NCHUNK, chunk_body, 0)


def kernel(x, token_table, pos_table):
    xr = x.astype(jnp.int32).reshape(NW * NCHUNK, NGROUP, GROUP)
    out = _embed_sc(xr, token_table, pos_table)
    return out.reshape(BATCH, MAXLEN, EMBED)
